# baseline (speedup 1.0000x reference)
import functools

import jax
import jax.numpy as jnp
from jax import lax
from jax.experimental import pallas as pl
from jax.experimental.pallas import tpu as pltpu
from jax.experimental.pallas import tpu_sc as plsc

D_MODEL = 64
LANES = 16
NUM_CORES = 2
NUM_SUBCORES = 16
NUM_WORKERS = NUM_CORES * NUM_SUBCORES


@functools.lru_cache(maxsize=None)
def _build(B: int, T: int, V: int, D: int):
    assert T % NUM_WORKERS == 0 and D % LANES == 0
    CH = T // NUM_WORKERS
    assert CH % 8 == 0 and CH <= 128
    mesh = plsc.VectorSubcoreMesh(core_axis_name="c", subcore_axis_name="s")

    @functools.partial(
        pl.kernel,
        mesh=mesh,
        compiler_params=pltpu.CompilerParams(use_tc_tiling_on_sc=False),
        out_type=jax.ShapeDtypeStruct((B, T, D), jnp.float32),
        scratch_types=[
            pltpu.VMEM((B, CH), jnp.int32),
            pltpu.VMEM((CH, D), jnp.float32),
            pltpu.VMEM((2, CH, D), jnp.float32),
            pltpu.SemaphoreType.DMA,
            pltpu.SemaphoreType.DMA,
        ],
    )
    def k(x_hbm, tok_hbm, pos_hbm, out_hbm, idx_v, pos_v, rows_v, sem0, sem1):
        w = lax.axis_index("s") * NUM_CORES + lax.axis_index("c")
        t0 = w * CH
        pltpu.sync_copy(pos_hbm.at[pl.ds(t0, CH)], pos_v)
        pltpu.sync_copy(x_hbm.at[w], idx_v)

        sems = [sem0, sem1]
        handles = [None, None]

        def start(b):
            buf = b % 2
            handles[buf] = pltpu.async_copy(
                tok_hbm.at[idx_v.at[b]], rows_v.at[buf], sems[buf])

        start(0)
        for b in range(B):
            buf = b % 2
            if b + 1 < B:
                start(b + 1)
            handles[buf].wait()
            rows = rows_v.at[buf]

            def body(r, carry):
                for kk in range(D // LANES):
                    sl = pl.ds(kk * LANES, LANES)
                    rows[r, sl] = rows[r, sl] + pos_v[r, sl]
                return carry

            lax.fori_loop(0, CH, body, 0)
            pltpu.sync_copy(rows, out_hbm.at[b, pl.ds(t0, CH)])

    return k


def kernel(x, tok_emb, pos_emb):
    B, T = x.shape
    V, D = tok_emb.shape
    k = _build(B, T, V, D)
    ch = T // NUM_WORKERS
    xr = x.astype(jnp.int32).reshape(B, NUM_WORKERS, ch).transpose(1, 0, 2)
    return k(xr, tok_emb, pos_emb)
